# baseline (speedup 1.0000x reference)
import functools

import jax
import jax.numpy as jnp
from jax import lax
from jax.experimental import pallas as pl
from jax.experimental.pallas import tpu as pltpu
from jax.experimental.pallas import tpu_sc as plsc

_NC = 2
_NS = 16
_NW = _NC * _NS

_ROWS = 4096
_NB = 8
_F = 128
_CH = 16
_PER_W = _ROWS // _NW
_NCHUNK = _PER_W // _CH


def _sc_body(x1_hbm, x2_hbm, out_hbm, a_v, b_v, lsem, wsem):
    wid = lax.axis_index("s") * _NC + lax.axis_index("c")
    base = wid * _PER_W

    def start_loads(c):
        s = c % 2
        bk = base + c * _CH
        return [
            pltpu.async_copy(x1_hbm.at[pl.ds(bk, _CH)], a_v.at[s], lsem),
            pltpu.async_copy(x2_hbm.at[pl.ds(bk, _CH)], b_v.at[s], lsem),
        ]

    loads = {0: start_loads(0)}
    writes = {}
    for c in range(_NCHUNK):
        s = c % 2
        if c >= 1:
            for d in writes[c - 1]:
                d.wait()
        if c + 1 < _NCHUNK:
            loads[c + 1] = start_loads(c + 1)
        for d in loads[c]:
            d.wait()
        bk = base + c * _CH
        ws = []
        for i in range(_NB):
            ws.append(pltpu.async_copy(
                a_v.at[s], out_hbm.at[pl.ds(bk, _CH), i, :, pl.ds(0, _F)],
                wsem))
        for j in range(_NB):
            ws.append(pltpu.async_copy(
                b_v.at[s], out_hbm.at[pl.ds(bk, _CH), :, j, pl.ds(_F, _F)],
                wsem))
        writes[c] = ws
    for d in writes[_NCHUNK - 1]:
        d.wait()


_sc_assemble = functools.partial(
    pl.kernel,
    out_type=jax.ShapeDtypeStruct((_ROWS, _NB, _NB, 2 * _F), jnp.float32),
    mesh=plsc.VectorSubcoreMesh(core_axis_name="c", subcore_axis_name="s"),
    scratch_types=[
        pltpu.VMEM((2, _CH, _NB, _F), jnp.float32),
        pltpu.VMEM((2, _CH, _NB, _F), jnp.float32),
        pltpu.SemaphoreType.DMA,
        pltpu.SemaphoreType.DMA,
    ],
)(_sc_body)


def kernel(x1, x2):
    nbatch, nk, nb, f = x1.shape
    rows = nbatch * nk
    out = _sc_assemble(x1.reshape(rows, nb, f), x2.reshape(rows, nb, f))
    return out.reshape(nbatch, nk, nb, nb, 2 * f)
